# Initial kernel scaffold; baseline (speedup 1.0000x reference)
import jax
import jax.numpy as jnp
from jax.experimental import pallas as pl


def kernel(z, edge):
    raise NotImplementedError("write your pallas kernel here")


# speedup vs baseline: 1.8252x; 1.8252x over previous
import functools

import jax
import jax.numpy as jnp
from jax import lax
from jax.experimental import pallas as pl
from jax.experimental.pallas import tpu as pltpu


def _round_up(x, m):
    return (x + m - 1) // m * m


def _edge_score_kernel(zt_ref, ids_ref, o_ref, *, d):
    n = zt_ref.shape[1]
    te = ids_ref.shape[2]

    ids = ids_ref[0].reshape(1, 2 * te)
    node_iota = lax.broadcasted_iota(jnp.int32, (n, 2 * te), 0)
    oh = (node_iota == ids).astype(zt_ref.dtype)

    g = jnp.dot(zt_ref[...], oh, preferred_element_type=jnp.float32)

    s = jnp.sum(g[:, :te] * g[:, te:], axis=0, keepdims=True)
    o_ref[0] = jax.nn.sigmoid(s)


@functools.partial(jax.jit, static_argnames=("tile_e",))
def _dot_edge_scores(z, edge, tile_e=1024):
    n, d = z.shape
    e = edge.shape[1]
    zt = z.T

    te = min(tile_e, _round_up(e, 128))
    e_pad = _round_up(e, te)
    ids = edge.astype(jnp.int32)
    if e_pad != e:
        ids = jnp.pad(ids, ((0, 0), (0, e_pad - e)))
    g = e_pad // te
    ids = ids.reshape(2, g, te).transpose(1, 0, 2)

    kernel_fn = functools.partial(_edge_score_kernel, d=d)
    cost = pl.CostEstimate(
        flops=4 * e_pad * n * d + 2 * e_pad * d,
        transcendentals=e_pad,
        bytes_accessed=d * n * 4 + 2 * e_pad * 4 + e_pad * 4,
    )
    out = pl.pallas_call(
        kernel_fn,
        out_shape=jax.ShapeDtypeStruct((g, 1, te), jnp.float32),
        grid=(g,),
        in_specs=[
            pl.BlockSpec((d, n), lambda i: (0, 0)),
            pl.BlockSpec((1, 2, te), lambda i: (i, 0, 0)),
        ],
        out_specs=pl.BlockSpec((1, 1, te), lambda i: (i, 0, 0)),
        compiler_params=pltpu.CompilerParams(
            dimension_semantics=("parallel",),
        ),
        cost_estimate=cost,
    )(zt, ids)
    return out.reshape(e_pad)[:e]


def kernel(z, edge):
    return _dot_edge_scores(z, edge)


# speedup vs baseline: 2.1271x; 1.1654x over previous
import functools

import jax
import jax.numpy as jnp
from jax import lax
from jax.experimental import pallas as pl
from jax.experimental.pallas import tpu as pltpu


def _round_up(x, m):
    return (x + m - 1) // m * m


def _edge_score_kernel(zt_ref, ids_ref, o_ref, *, d):
    n = zt_ref.shape[1]
    te2 = ids_ref.shape[2]
    te = te2 // 2

    ids = ids_ref[0]
    node_iota = lax.broadcasted_iota(jnp.int32, (n, te2), 0)
    oh = (node_iota == ids).astype(zt_ref.dtype)

    g = jnp.dot(zt_ref[...], oh, preferred_element_type=jnp.float32)

    s = jnp.sum(g[:, :te] * g[:, te:], axis=0, keepdims=True)
    o_ref[0] = jax.nn.sigmoid(s)


@functools.partial(jax.jit, static_argnames=("tile_e",))
def _dot_edge_scores(z, edge, tile_e=8192):
    n, d = z.shape
    e = edge.shape[1]
    zt = z.T

    te = min(tile_e, _round_up(e, 128))
    e_pad = _round_up(e, te)
    ids = edge.astype(jnp.int32)
    if e_pad != e:
        ids = jnp.pad(ids, ((0, 0), (0, e_pad - e)))
    g = e_pad // te
    ids = ids.reshape(2, g, te).transpose(1, 0, 2).reshape(g, 1, 2 * te)

    kernel_fn = functools.partial(_edge_score_kernel, d=d)
    cost = pl.CostEstimate(
        flops=4 * e_pad * n * d + 2 * e_pad * d,
        transcendentals=e_pad,
        bytes_accessed=d * n * 4 + 2 * e_pad * 4 + e_pad * 4,
    )
    out = pl.pallas_call(
        kernel_fn,
        out_shape=jax.ShapeDtypeStruct((g, 1, te), jnp.float32),
        grid=(g,),
        in_specs=[
            pl.BlockSpec((d, n), lambda i: (0, 0)),
            pl.BlockSpec((1, 1, 2 * te), lambda i: (i, 0, 0)),
        ],
        out_specs=pl.BlockSpec((1, 1, te), lambda i: (i, 0, 0)),
        compiler_params=pltpu.CompilerParams(
            dimension_semantics=("parallel",),
        ),
        cost_estimate=cost,
    )(zt, ids)
    return out.reshape(e_pad)[:e]


def kernel(z, edge):
    return _dot_edge_scores(z, edge)
